# Initial kernel scaffold; baseline (speedup 1.0000x reference)
import jax
import jax.numpy as jnp
from jax.experimental import pallas as pl


def kernel(com_maps, pairwise_t, dir_mask, W1, b1, W2, b2, W3, b3):
    raise NotImplementedError("write your pallas kernel here")


# speedup vs baseline: 1.4367x; 1.4367x over previous
import functools

import jax
import jax.numpy as jnp
from jax import lax
from jax.experimental import pallas as pl
from jax.experimental.pallas import tpu as pltpu
from jax.experimental.pallas import tpu_sc as plsc

_B = 4
_C = 2
_H = 100
_W = 252
_IN = _C * _H * _W
_HID = 512
_K_TOP = int(0.2 * _H * _W)
_INT_MIN = -(2 ** 31)

_BM = 32
_BMQ = _BM // 4
_NB1 = _HID // _BM
_BN = 1440
_BNQ = _BN // 4
_NB3 = _IN // _BN

_ROWS_MAIN = 3152
_ROWS_LAST = _IN - 15 * _ROWS_MAIN
_WMAIN = _ROWS_MAIN * 4
_WLAST = _ROWS_LAST * 4


def _mlp_body(cm, pt, dm, w1a, w1b, w1c, w1d, b1, w2, b2,
              w3a, w3b, w3c, w3d, b3, out, xT, h1, h2):
    i = pl.program_id(0)

    @pl.when(i == 0)
    def _():
        xT[...] = (cm[...] * dm[...] + pt[...]).T

    @pl.when(i < _NB1)
    def _():
        for q, wq in enumerate((w1a, w1b, w1c, w1d)):
            acc = jnp.dot(wq[...], xT[...], preferred_element_type=jnp.float32)
            h1[pl.ds(i * _BM + q * _BMQ, _BMQ), :] = jnp.maximum(
                acc + b1[pl.ds(q * _BMQ, _BMQ), :], 0.0)

    @pl.when(i == _NB1)
    def _():
        a = jnp.dot(w2[...], h1[...], preferred_element_type=jnp.float32)
        h2[...] = jnp.maximum(a + b2[...], 0.0)

    @pl.when(i >= _NB1)
    def _():
        for q, wq in enumerate((w3a, w3b, w3c, w3d)):
            acc = jnp.dot(wq[...], h2[...], preferred_element_type=jnp.float32)
            out[pl.ds(q * _BNQ, _BNQ), :] = acc + b3[pl.ds(q * _BNQ, _BNQ), :]


def _sc_select_body(logits_hbm, out_hbm, vals, ub, hist, merged, allsl,
                    outbuf0, outbuf1, slots):
    c = lax.axis_index("c")
    s = lax.axis_index("s")
    lanes = jnp.arange(16, dtype=jnp.int32)
    lane_b = lanes & 3
    lane_bsub = lane_b & 1
    active = (lane_b >> 1) == c
    iminv = jnp.int32(_INT_MIN)

    nv = jnp.where(s == 15, _WLAST // 16, _WMAIN // 16)
    wbase = s * _WMAIN

    @pl.when(s < 15)
    def _():
        pltpu.sync_copy(logits_hbm.at[pl.ds(wbase, _WMAIN)],
                        vals.at[pl.ds(0, _WMAIN)])

    @pl.when(s == 15)
    def _():
        pltpu.sync_copy(logits_hbm.at[pl.ds(wbase, _WLAST)],
                        vals.at[pl.ds(0, _WLAST)])

    def _mkub(i, carry):
        x = vals[pl.ds(i * 16, 16)]
        ib = lax.bitcast_convert_type(x, jnp.int32)
        key = jnp.where(ib >= 0, ib,
                        jnp.bitwise_xor(jnp.bitwise_not(ib), iminv))
        ub[pl.ds(i * 16, 16)] = jnp.bitwise_xor(key, iminv)
        return carry

    lax.fori_loop(0, nv, _mkub, 0)

    selv = jnp.zeros((16,), jnp.int32)
    krem0 = jnp.int32(_K_TOP)
    krem1 = jnp.int32(_K_TOP)
    ones_i = jnp.ones((16,), jnp.int32)
    zv = jnp.zeros((16,), jnp.int32)

    for rnd in range(4):
        shift = 24 - 8 * rnd

        def _zero(i, carry):
            hist[pl.ds(i * 16, 16)] = zv
            return carry

        lax.fori_loop(0, 256, _zero, 0)
        selv_c = selv

        def _histpass(i, carry, shift=shift, rnd=rnd, selv_c=selv_c):
            ubv = ub[pl.ds(i * 16, 16)]
            byte = lax.shift_right_logical(ubv, shift) & 255
            if rnd == 0:
                pmask = active
            else:
                pref_eq = lax.shift_right_logical(ubv, shift + 8) == \
                    lax.shift_right_logical(selv_c, shift + 8)
                pmask = active & pref_eq
            plsc.addupdate_scatter(hist, [lanes * 256 + byte], ones_i,
                                   mask=pmask)
            return carry

        lax.fori_loop(0, nv, _histpass, 0)

        for bsub in range(2):
            b = 2 * c + bsub
            for j in range(16):
                h = hist[pl.ds(b * 256 + j * 16, 16)]
                h = h + hist[pl.ds((b + 4) * 256 + j * 16, 16)]
                h = h + hist[pl.ds((b + 8) * 256 + j * 16, 16)]
                h = h + hist[pl.ds((b + 12) * 256 + j * 16, 16)]
                merged[pl.ds(bsub * 256 + j * 16, 16)] = h

        pltpu.sync_copy(merged, slots.at[pl.ds(rnd * 8192 + s * 512, 512)])
        plsc.subcore_barrier()
        pltpu.sync_copy(slots.at[pl.ds(rnd * 8192, 8192)], allsl)
        for j in range(32):
            acc = allsl[pl.ds(j * 16, 16)]
            for t in range(1, 16):
                acc = acc + allsl[pl.ds(t * 512 + j * 16, 16)]
            merged[pl.ds(j * 16, 16)] = acc

        sbytes = []
        krems = []
        for bsub, krem in ((0, krem0), (1, krem1)):
            hsums = [jnp.sum(merged[pl.ds(bsub * 256 + j * 16, 16)])
                     for j in range(16)]
            cum = jnp.int32(0)
            found = jnp.bool_(False)
            jstar = jnp.int32(0)
            cum_above = jnp.int32(0)
            for j in range(15, -1, -1):
                ncum = cum + hsums[j]
                hit = jnp.logical_and(jnp.logical_not(found), ncum >= krem)
                jstar = jnp.where(hit, j, jstar)
                cum_above = jnp.where(hit, cum, cum_above)
                found = jnp.logical_or(found, hit)
                cum = ncum
            hv = merged[pl.ds(bsub * 256 + jstar * 16, 16)]
            cs = plsc.cumsum(lax.rev(hv, (0,)))
            hitm = (cum_above + cs) >= krem
            mstar = plsc.all_reduce_ffs(hitm)
            c_at = jnp.sum(jnp.where(lanes == mstar, cs, 0))
            h_at = jnp.sum(jnp.where(lanes == (15 - mstar), hv, 0))
            sbytes.append(jstar * 16 + 15 - mstar)
            krems.append(krem - cum_above - (c_at - h_at))

        krem0, krem1 = krems
        selbyte = jnp.where(lane_bsub == 0, sbytes[0], sbytes[1])
        selv = selv | lax.shift_left(selbyte, shift)

    thrv = jnp.bitwise_xor(selv, iminv)
    m0 = active & (lane_bsub == 0)
    m1 = active & (lane_bsub == 1)

    def _maskpass(i, carry):
        ubv = ub[pl.ds(i * 16, 16)]
        keyv = jnp.bitwise_xor(ubv, iminv)
        ge = (keyv > thrv) | (ubv == selv)
        outv = jnp.where(ge, 1.0, 0.0).astype(jnp.float32)
        col = i * 4 + lax.shift_right_logical(lanes, 2)
        plsc.store_scatter(outbuf0, [col], outv, mask=m0)
        plsc.store_scatter(outbuf1, [col], outv, mask=m1)
        return carry

    lax.fori_loop(0, nv, _maskpass, 0)

    rbase = s * _ROWS_MAIN
    for bsub, ob in ((0, outbuf0), (1, outbuf1)):
        b = 2 * c + bsub

        @pl.when(s < 15)
        def _(b=b, ob=ob):
            pltpu.sync_copy(ob.at[pl.ds(0, _ROWS_MAIN)],
                            out_hbm.at[pl.ds(b * _IN + rbase, _ROWS_MAIN)])

        @pl.when(s == 15)
        def _(b=b, ob=ob):
            pltpu.sync_copy(ob.at[pl.ds(0, _ROWS_LAST)],
                            out_hbm.at[pl.ds(b * _IN + rbase, _ROWS_LAST)])


def _sc_select(logits_flat):
    mesh = plsc.VectorSubcoreMesh(core_axis_name="c", subcore_axis_name="s")
    kfn = functools.partial(
        pl.kernel,
        mesh=mesh,
        compiler_params=pltpu.CompilerParams(needs_layout_passes=False),
        out_type=jax.ShapeDtypeStruct((_B * _IN,), jnp.float32),
        scratch_types=[
            pltpu.VMEM((_WMAIN,), jnp.float32),
            pltpu.VMEM((_WMAIN,), jnp.int32),
            pltpu.VMEM((4096,), jnp.int32),
            pltpu.VMEM((512,), jnp.int32),
            pltpu.VMEM((8192,), jnp.int32),
            pltpu.VMEM((_ROWS_MAIN,), jnp.float32),
            pltpu.VMEM((_ROWS_MAIN,), jnp.float32),
            pltpu.VMEM_SHARED((4 * 8192,), jnp.int32),
        ],
    )(_sc_select_body)
    return kfn(logits_flat)


def kernel(com_maps, pairwise_t, dir_mask, W1, b1, W2, b2, W3, b3):
    cm = com_maps.reshape(_B, _IN)
    pt = pairwise_t.reshape(_B, _IN)
    dm = dir_mask.reshape(_B, _IN)
    b1c = b1.reshape(_HID, 1)
    b2c = b2.reshape(_HID, 1)
    b3c = b3.reshape(_IN, 1)

    logits = pl.pallas_call(
        _mlp_body,
        grid=(_NB1 + _NB3,),
        in_specs=[
            pl.BlockSpec((_B, _IN), lambda i: (0, 0)),
            pl.BlockSpec((_B, _IN), lambda i: (0, 0)),
            pl.BlockSpec((_B, _IN), lambda i: (0, 0)),
        ] + [
            pl.BlockSpec((_BMQ, _IN),
                         (lambda q: lambda i: (jnp.minimum(i, _NB1 - 1) * 4 + q, 0))(q))
            for q in range(4)
        ] + [
            pl.BlockSpec((_BM, 1), lambda i: (jnp.minimum(i, _NB1 - 1), 0)),
            pl.BlockSpec((_HID, _HID), lambda i: (0, 0)),
            pl.BlockSpec((_HID, 1), lambda i: (0, 0)),
        ] + [
            pl.BlockSpec((_BNQ, _HID),
                         (lambda q: lambda i: (jnp.maximum(i - _NB1, 0) * 4 + q, 0))(q))
            for q in range(4)
        ] + [
            pl.BlockSpec((_BN, 1), lambda i: (jnp.maximum(i - _NB1, 0), 0)),
        ],
        out_specs=pl.BlockSpec((_BN, _B), lambda i: (jnp.maximum(i - _NB1, 0), 0)),
        out_shape=jax.ShapeDtypeStruct((_IN, _B), jnp.float32),
        scratch_shapes=[
            pltpu.VMEM((_IN, _B), jnp.float32),
            pltpu.VMEM((_HID, _B), jnp.float32),
            pltpu.VMEM((_HID, _B), jnp.float32),
        ],
    )(cm, pt, dm, W1, W1, W1, W1, b1c, W2, b2c, W3, W3, W3, W3, b3c)

    mask = _sc_select(logits.reshape(-1))
    return mask.reshape(_B, _C, _H, _W)


# speedup vs baseline: 1.8177x; 1.2652x over previous
import functools

import jax
import jax.numpy as jnp
from jax import lax
from jax.experimental import pallas as pl
from jax.experimental.pallas import tpu as pltpu
from jax.experimental.pallas import tpu_sc as plsc

_B = 4
_C = 2
_H = 100
_W = 252
_IN = _C * _H * _W
_HID = 512
_K_TOP = int(0.2 * _H * _W)
_INT_MIN = -(2 ** 31)

_BM = 32
_BMQ = _BM // 4
_NB1 = _HID // _BM
_BN = 1440
_BNQ = _BN // 4
_NB3 = _IN // _BN

_ROWS_MAIN = 3152
_ROWS_LAST = _IN - 15 * _ROWS_MAIN
_WMAIN = _ROWS_MAIN * 4
_WLAST = _ROWS_LAST * 4


_DN = (((1,), (1,)), ((), ()))


def _mlp_body(cm, pt, dm, w1a, w1b, w1c, w1d, b1, w2, b2,
              w3a, w3b, w3c, w3d, b3, out, xs, h1, h2):
    i = pl.program_id(0)

    @pl.when(i == 0)
    def _():
        xs[...] = cm[...] * dm[...] + pt[...]

    @pl.when(i < _NB1)
    def _():
        for q, wq in enumerate((w1a, w1b, w1c, w1d)):
            acc = lax.dot_general(wq[...], xs[...], _DN,
                                  preferred_element_type=jnp.float32)
            h1[pl.ds(i * _BM + q * _BMQ, _BMQ), :] = jnp.maximum(
                acc + b1[pl.ds(q * _BMQ, _BMQ), :], 0.0)

    @pl.when(i == _NB1)
    def _():
        a = jnp.dot(w2[...], h1[...], preferred_element_type=jnp.float32)
        h2[...] = jnp.maximum(a + b2[...], 0.0)

    @pl.when(i >= _NB1)
    def _():
        for q, wq in enumerate((w3a, w3b, w3c, w3d)):
            acc = jnp.dot(wq[...], h2[...], preferred_element_type=jnp.float32)
            out[pl.ds(q * _BNQ, _BNQ), :] = acc + b3[pl.ds(q * _BNQ, _BNQ), :]


def _sc_select_body(logits_hbm, out_hbm, vals, ub, hist, merged, allsl,
                    outbuf0, outbuf1, slots):
    c = lax.axis_index("c")
    s = lax.axis_index("s")
    lanes = jnp.arange(16, dtype=jnp.int32)
    lane_b = lanes & 3
    lane_bsub = lane_b & 1
    active = (lane_b >> 1) == c
    iminv = jnp.int32(_INT_MIN)

    nv = jnp.where(s == 15, _WLAST // 16, _WMAIN // 16)
    wbase = s * _WMAIN

    @pl.when(s < 15)
    def _():
        pltpu.sync_copy(logits_hbm.at[pl.ds(wbase, _WMAIN)],
                        vals.at[pl.ds(0, _WMAIN)])

    @pl.when(s == 15)
    def _():
        pltpu.sync_copy(logits_hbm.at[pl.ds(wbase, _WLAST)],
                        vals.at[pl.ds(0, _WLAST)])

    selv = jnp.zeros((16,), jnp.int32)
    krem0 = jnp.int32(_K_TOP)
    krem1 = jnp.int32(_K_TOP)
    ones_i = jnp.ones((16,), jnp.int32)
    zv = jnp.zeros((16,), jnp.int32)

    for rnd in range(4):
        shift = 24 - 8 * rnd

        def _zero(i, carry):
            hist[pl.ds(i * 16, 16)] = zv
            return carry

        lax.fori_loop(0, 256, _zero, 0)
        selv_c = selv

        def _histpass(i, carry, shift=shift, rnd=rnd, selv_c=selv_c):
            if rnd == 0:
                x = vals[pl.ds(i * 16, 16)]
                ib = lax.bitcast_convert_type(x, jnp.int32)
                key = jnp.where(ib >= 0, ib,
                                jnp.bitwise_xor(jnp.bitwise_not(ib), iminv))
                ubv = jnp.bitwise_xor(key, iminv)
                ub[pl.ds(i * 16, 16)] = ubv
                pmask = active
            else:
                ubv = ub[pl.ds(i * 16, 16)]
                pref_eq = lax.shift_right_logical(ubv, shift + 8) == \
                    lax.shift_right_logical(selv_c, shift + 8)
                pmask = active & pref_eq
            byte = lax.shift_right_logical(ubv, shift) & 255
            plsc.addupdate_scatter(hist, [lanes * 256 + byte], ones_i,
                                   mask=pmask)
            return carry

        lax.fori_loop(0, nv, _histpass, 0)

        for bsub in range(2):
            b = 2 * c + bsub
            for j in range(16):
                h = hist[pl.ds(b * 256 + j * 16, 16)]
                h = h + hist[pl.ds((b + 4) * 256 + j * 16, 16)]
                h = h + hist[pl.ds((b + 8) * 256 + j * 16, 16)]
                h = h + hist[pl.ds((b + 12) * 256 + j * 16, 16)]
                merged[pl.ds(bsub * 256 + j * 16, 16)] = h

        pltpu.sync_copy(merged, slots.at[pl.ds(rnd * 8192 + s * 512, 512)])
        plsc.subcore_barrier()
        pltpu.sync_copy(slots.at[pl.ds(rnd * 8192, 8192)], allsl)
        for j in range(32):
            acc = allsl[pl.ds(j * 16, 16)]
            for t in range(1, 16):
                acc = acc + allsl[pl.ds(t * 512 + j * 16, 16)]
            merged[pl.ds(j * 16, 16)] = acc

        sbytes = []
        krems = []
        for bsub, krem in ((0, krem0), (1, krem1)):
            hsums = [jnp.sum(merged[pl.ds(bsub * 256 + j * 16, 16)])
                     for j in range(16)]
            cum = jnp.int32(0)
            found = jnp.bool_(False)
            jstar = jnp.int32(0)
            cum_above = jnp.int32(0)
            for j in range(15, -1, -1):
                ncum = cum + hsums[j]
                hit = jnp.logical_and(jnp.logical_not(found), ncum >= krem)
                jstar = jnp.where(hit, j, jstar)
                cum_above = jnp.where(hit, cum, cum_above)
                found = jnp.logical_or(found, hit)
                cum = ncum
            hv = merged[pl.ds(bsub * 256 + jstar * 16, 16)]
            cs = plsc.cumsum(lax.rev(hv, (0,)))
            hitm = (cum_above + cs) >= krem
            mstar = plsc.all_reduce_ffs(hitm)
            c_at = jnp.sum(jnp.where(lanes == mstar, cs, 0))
            h_at = jnp.sum(jnp.where(lanes == (15 - mstar), hv, 0))
            sbytes.append(jstar * 16 + 15 - mstar)
            krems.append(krem - cum_above - (c_at - h_at))

        krem0, krem1 = krems
        selbyte = jnp.where(lane_bsub == 0, sbytes[0], sbytes[1])
        selv = selv | lax.shift_left(selbyte, shift)

    thrv = jnp.bitwise_xor(selv, iminv)
    m0 = active & (lane_bsub == 0)
    m1 = active & (lane_bsub == 1)

    def _maskpass(i, carry):
        ubv = ub[pl.ds(i * 16, 16)]
        keyv = jnp.bitwise_xor(ubv, iminv)
        ge = (keyv > thrv) | (ubv == selv)
        outv = jnp.where(ge, 1.0, 0.0).astype(jnp.float32)
        col = i * 4 + lax.shift_right_logical(lanes, 2)
        plsc.store_scatter(outbuf0, [col], outv, mask=m0)
        plsc.store_scatter(outbuf1, [col], outv, mask=m1)
        return carry

    lax.fori_loop(0, nv, _maskpass, 0)

    rbase = s * _ROWS_MAIN
    for bsub, ob in ((0, outbuf0), (1, outbuf1)):
        b = 2 * c + bsub

        @pl.when(s < 15)
        def _(b=b, ob=ob):
            pltpu.sync_copy(ob.at[pl.ds(0, _ROWS_MAIN)],
                            out_hbm.at[pl.ds(b * _IN + rbase, _ROWS_MAIN)])

        @pl.when(s == 15)
        def _(b=b, ob=ob):
            pltpu.sync_copy(ob.at[pl.ds(0, _ROWS_LAST)],
                            out_hbm.at[pl.ds(b * _IN + rbase, _ROWS_LAST)])


def _sc_select(logits_flat):
    mesh = plsc.VectorSubcoreMesh(core_axis_name="c", subcore_axis_name="s")
    kfn = functools.partial(
        pl.kernel,
        mesh=mesh,
        compiler_params=pltpu.CompilerParams(needs_layout_passes=False),
        out_type=jax.ShapeDtypeStruct((_B * _IN,), jnp.float32),
        scratch_types=[
            pltpu.VMEM((_WMAIN,), jnp.float32),
            pltpu.VMEM((_WMAIN,), jnp.int32),
            pltpu.VMEM((4096,), jnp.int32),
            pltpu.VMEM((512,), jnp.int32),
            pltpu.VMEM((8192,), jnp.int32),
            pltpu.VMEM((_ROWS_MAIN,), jnp.float32),
            pltpu.VMEM((_ROWS_MAIN,), jnp.float32),
            pltpu.VMEM_SHARED((4 * 8192,), jnp.int32),
        ],
    )(_sc_select_body)
    return kfn(logits_flat)


def kernel(com_maps, pairwise_t, dir_mask, W1, b1, W2, b2, W3, b3):
    cm = com_maps.reshape(_B, _IN)
    pt = pairwise_t.reshape(_B, _IN)
    dm = dir_mask.reshape(_B, _IN)
    b1c = b1.reshape(_HID, 1)
    b2c = b2.reshape(_HID, 1)
    b3c = b3.reshape(_IN, 1)

    logits = pl.pallas_call(
        _mlp_body,
        grid=(_NB1 + _NB3,),
        in_specs=[
            pl.BlockSpec((_B, _IN), lambda i: (0, 0)),
            pl.BlockSpec((_B, _IN), lambda i: (0, 0)),
            pl.BlockSpec((_B, _IN), lambda i: (0, 0)),
        ] + [
            pl.BlockSpec((_BMQ, _IN),
                         (lambda q: lambda i: (jnp.minimum(i, _NB1 - 1) * 4 + q, 0))(q))
            for q in range(4)
        ] + [
            pl.BlockSpec((_BM, 1), lambda i: (jnp.minimum(i, _NB1 - 1), 0)),
            pl.BlockSpec((_HID, _HID), lambda i: (0, 0)),
            pl.BlockSpec((_HID, 1), lambda i: (0, 0)),
        ] + [
            pl.BlockSpec((_BNQ, _HID),
                         (lambda q: lambda i: (jnp.maximum(i - _NB1, 0) * 4 + q, 0))(q))
            for q in range(4)
        ] + [
            pl.BlockSpec((_BN, 1), lambda i: (jnp.maximum(i - _NB1, 0), 0)),
        ],
        out_specs=pl.BlockSpec((_BN, _B), lambda i: (jnp.maximum(i - _NB1, 0), 0)),
        out_shape=jax.ShapeDtypeStruct((_IN, _B), jnp.float32),
        scratch_shapes=[
            pltpu.VMEM((_B, _IN), jnp.float32),
            pltpu.VMEM((_HID, _B), jnp.float32),
            pltpu.VMEM((_HID, _B), jnp.float32),
        ],
    )(cm, pt, dm, W1, W1, W1, W1, b1c, W2, b2c, W3, W3, W3, W3, b3c)

    mask = _sc_select(logits.reshape(-1))
    return mask.reshape(_B, _C, _H, _W)


# speedup vs baseline: 1.8510x; 1.0183x over previous
import functools

import jax
import jax.numpy as jnp
from jax import lax
from jax.experimental import pallas as pl
from jax.experimental.pallas import tpu as pltpu
from jax.experimental.pallas import tpu_sc as plsc

_B = 4
_C = 2
_H = 100
_W = 252
_IN = _C * _H * _W
_HID = 512
_K_TOP = int(0.2 * _H * _W)
_INT_MIN = -(2 ** 31)

_ROWS_MAIN = 3152
_ROWS_LAST = _IN - 15 * _ROWS_MAIN
_WMAIN = _ROWS_MAIN * 4
_WLAST = _ROWS_LAST * 4


def _fc1_body(cm, pt, dm, w1, b1, h1_out, xT):
    i = pl.program_id(0)

    @pl.when(i == 0)
    def _():
        xT[...] = (cm[...] * dm[...] + pt[...]).T

    acc = jnp.dot(w1[...], xT[...], preferred_element_type=jnp.float32)
    h1_out[...] = jnp.maximum(acc + b1[...], 0.0)


def _fc23_body(h1, w2, b2, w3, b3, out, h2):
    i = pl.program_id(0)

    @pl.when(i == 0)
    def _():
        a = jnp.dot(w2[...], h1[...], preferred_element_type=jnp.float32)
        h2[...] = jnp.maximum(a + b2[...], 0.0)

    out[...] = jnp.dot(w3[...], h2[...], preferred_element_type=jnp.float32) + b3[...]


def _sc_select_body(logits_hbm, out_hbm, vals, ub, hist, merged, allsl,
                    outbuf0, outbuf1, slots):
    c = lax.axis_index("c")
    s = lax.axis_index("s")
    lanes = jnp.arange(16, dtype=jnp.int32)
    lane_b = lanes & 3
    lane_bsub = lane_b & 1
    active = (lane_b >> 1) == c
    iminv = jnp.int32(_INT_MIN)

    nv4 = jnp.where(s == 15, _WLAST // 64, _WMAIN // 64)
    wbase = s * _WMAIN

    @pl.when(s < 15)
    def _():
        pltpu.sync_copy(logits_hbm.at[pl.ds(wbase, _WMAIN)],
                        vals.at[pl.ds(0, _WMAIN)])

    @pl.when(s == 15)
    def _():
        pltpu.sync_copy(logits_hbm.at[pl.ds(wbase, _WLAST)],
                        vals.at[pl.ds(0, _WLAST)])

    selv = jnp.zeros((16,), jnp.int32)
    krem0 = jnp.int32(_K_TOP)
    krem1 = jnp.int32(_K_TOP)
    ones_i = jnp.ones((16,), jnp.int32)
    zv = jnp.zeros((16,), jnp.int32)

    for rnd in range(4):
        shift = 24 - 8 * rnd

        def _zero(i, carry):
            for u in range(4):
                hist[pl.ds(i * 64 + u * 16, 16)] = zv
            return carry

        lax.fori_loop(0, 64, _zero, 0)
        selv_c = selv

        def _histpass(i, carry, shift=shift, rnd=rnd, selv_c=selv_c):
            for u in range(4):
                o = i * 64 + u * 16
                if rnd == 0:
                    x = vals[pl.ds(o, 16)]
                    ib = lax.bitcast_convert_type(x, jnp.int32)
                    key = jnp.where(ib >= 0, ib,
                                    jnp.bitwise_xor(jnp.bitwise_not(ib), iminv))
                    ubv = jnp.bitwise_xor(key, iminv)
                    ub[pl.ds(o, 16)] = ubv
                    pmask = active
                else:
                    ubv = ub[pl.ds(o, 16)]
                    pref_eq = lax.shift_right_logical(ubv, shift + 8) == \
                        lax.shift_right_logical(selv_c, shift + 8)
                    pmask = active & pref_eq
                byte = lax.shift_right_logical(ubv, shift) & 255
                plsc.addupdate_scatter(hist, [lanes * 256 + byte], ones_i,
                                       mask=pmask)
            return carry

        lax.fori_loop(0, nv4, _histpass, 0)

        for bsub in range(2):
            b = 2 * c + bsub
            for j in range(16):
                h = hist[pl.ds(b * 256 + j * 16, 16)]
                h = h + hist[pl.ds((b + 4) * 256 + j * 16, 16)]
                h = h + hist[pl.ds((b + 8) * 256 + j * 16, 16)]
                h = h + hist[pl.ds((b + 12) * 256 + j * 16, 16)]
                merged[pl.ds(bsub * 256 + j * 16, 16)] = h

        pltpu.sync_copy(merged, slots.at[pl.ds(rnd * 8192 + s * 512, 512)])
        plsc.subcore_barrier()
        pltpu.sync_copy(slots.at[pl.ds(rnd * 8192, 8192)], allsl)
        for j in range(32):
            acc = allsl[pl.ds(j * 16, 16)]
            for t in range(1, 16):
                acc = acc + allsl[pl.ds(t * 512 + j * 16, 16)]
            merged[pl.ds(j * 16, 16)] = acc

        sbytes = []
        krems = []
        for bsub, krem in ((0, krem0), (1, krem1)):
            hsums = [jnp.sum(merged[pl.ds(bsub * 256 + j * 16, 16)])
                     for j in range(16)]
            cum = jnp.int32(0)
            found = jnp.bool_(False)
            jstar = jnp.int32(0)
            cum_above = jnp.int32(0)
            for j in range(15, -1, -1):
                ncum = cum + hsums[j]
                hit = jnp.logical_and(jnp.logical_not(found), ncum >= krem)
                jstar = jnp.where(hit, j, jstar)
                cum_above = jnp.where(hit, cum, cum_above)
                found = jnp.logical_or(found, hit)
                cum = ncum
            hv = merged[pl.ds(bsub * 256 + jstar * 16, 16)]
            cs = plsc.cumsum(lax.rev(hv, (0,)))
            hitm = (cum_above + cs) >= krem
            mstar = plsc.all_reduce_ffs(hitm)
            c_at = jnp.sum(jnp.where(lanes == mstar, cs, 0))
            h_at = jnp.sum(jnp.where(lanes == (15 - mstar), hv, 0))
            sbytes.append(jstar * 16 + 15 - mstar)
            krems.append(krem - cum_above - (c_at - h_at))

        krem0, krem1 = krems
        selbyte = jnp.where(lane_bsub == 0, sbytes[0], sbytes[1])
        selv = selv | lax.shift_left(selbyte, shift)

    thrv = jnp.bitwise_xor(selv, iminv)
    m0 = active & (lane_bsub == 0)
    m1 = active & (lane_bsub == 1)

    def _maskpass(i, carry):
        for u in range(4):
            o = i * 4 + u
            ubv = ub[pl.ds(o * 16, 16)]
            keyv = jnp.bitwise_xor(ubv, iminv)
            ge = (keyv > thrv) | (ubv == selv)
            outv = jnp.where(ge, 1.0, 0.0).astype(jnp.float32)
            col = o * 4 + lax.shift_right_logical(lanes, 2)
            plsc.store_scatter(outbuf0, [col], outv, mask=m0)
            plsc.store_scatter(outbuf1, [col], outv, mask=m1)
        return carry

    lax.fori_loop(0, nv4, _maskpass, 0)

    rbase = s * _ROWS_MAIN
    for bsub, ob in ((0, outbuf0), (1, outbuf1)):
        b = 2 * c + bsub

        @pl.when(s < 15)
        def _(b=b, ob=ob):
            pltpu.sync_copy(ob.at[pl.ds(0, _ROWS_MAIN)],
                            out_hbm.at[pl.ds(b * _IN + rbase, _ROWS_MAIN)])

        @pl.when(s == 15)
        def _(b=b, ob=ob):
            pltpu.sync_copy(ob.at[pl.ds(0, _ROWS_LAST)],
                            out_hbm.at[pl.ds(b * _IN + rbase, _ROWS_LAST)])


def _sc_select(logits_flat):
    mesh = plsc.VectorSubcoreMesh(core_axis_name="c", subcore_axis_name="s")
    kfn = functools.partial(
        pl.kernel,
        mesh=mesh,
        compiler_params=pltpu.CompilerParams(needs_layout_passes=False),
        out_type=jax.ShapeDtypeStruct((_B * _IN,), jnp.float32),
        scratch_types=[
            pltpu.VMEM((_WMAIN,), jnp.float32),
            pltpu.VMEM((_WMAIN,), jnp.int32),
            pltpu.VMEM((4096,), jnp.int32),
            pltpu.VMEM((512,), jnp.int32),
            pltpu.VMEM((8192,), jnp.int32),
            pltpu.VMEM((_ROWS_MAIN,), jnp.float32),
            pltpu.VMEM((_ROWS_MAIN,), jnp.float32),
            pltpu.VMEM_SHARED((4 * 8192,), jnp.int32),
        ],
    )(_sc_select_body)
    return kfn(logits_flat)


def kernel(com_maps, pairwise_t, dir_mask, W1, b1, W2, b2, W3, b3):
    cm = com_maps.reshape(_B, _IN)
    pt = pairwise_t.reshape(_B, _IN)
    dm = dir_mask.reshape(_B, _IN)
    b1c = b1.reshape(_HID, 1)
    b2c = b2.reshape(_HID, 1)
    b3c = b3.reshape(_IN, 1)

    bm = 64
    h1 = pl.pallas_call(
        _fc1_body,
        grid=(_HID // bm,),
        in_specs=[
            pl.BlockSpec((_B, _IN), lambda i: (0, 0)),
            pl.BlockSpec((_B, _IN), lambda i: (0, 0)),
            pl.BlockSpec((_B, _IN), lambda i: (0, 0)),
            pl.BlockSpec((bm, _IN), lambda i: (i, 0)),
            pl.BlockSpec((bm, 1), lambda i: (i, 0)),
        ],
        out_specs=pl.BlockSpec((bm, _B), lambda i: (i, 0)),
        out_shape=jax.ShapeDtypeStruct((_HID, _B), jnp.float32),
        scratch_shapes=[pltpu.VMEM((_IN, _B), jnp.float32)],
    )(cm, pt, dm, W1, b1c)

    bn = 2016
    logits = pl.pallas_call(
        _fc23_body,
        grid=(_IN // bn,),
        in_specs=[
            pl.BlockSpec((_HID, _B), lambda i: (0, 0)),
            pl.BlockSpec((_HID, _HID), lambda i: (0, 0)),
            pl.BlockSpec((_HID, 1), lambda i: (0, 0)),
            pl.BlockSpec((bn, _HID), lambda i: (i, 0)),
            pl.BlockSpec((bn, 1), lambda i: (i, 0)),
        ],
        out_specs=pl.BlockSpec((bn, _B), lambda i: (i, 0)),
        out_shape=jax.ShapeDtypeStruct((_IN, _B), jnp.float32),
        scratch_shapes=[pltpu.VMEM((_HID, _B), jnp.float32)],
    )(h1, W2, b2c, W3, b3c)

    mask = _sc_select(logits.reshape(-1))
    return mask.reshape(_B, _C, _H, _W)
